# baseline (speedup 1.0000x reference)
import functools

import jax
import jax.numpy as jnp
from jax import lax
from jax.experimental import pallas as pl
from jax.experimental.pallas import tpu as pltpu
from jax.experimental.pallas import tpu_sc as plsc

_EMBED = 20
_NSLICE = 48

_SLOT_EXTRA = 36
_CB_W3 = 64
_CB_BIAS = 96


def _sc_body(big_hbm, idxp_hbm, o_hbm, idxp_v, rows_v, o_v, sem):
    c = lax.axis_index("c")
    s = lax.axis_index("s")

    @pl.when(jnp.logical_and(c == 0, s == 0))
    def _():
        pltpu.sync_copy(idxp_hbm, idxp_v)

        lanes = lax.iota(jnp.int32, 16)
        r = [jnp.full((16,), k, jnp.int32) for k in range(7)]
        s0 = plsc.load_gather(idxp_v, [r[0], lanes])
        s1 = plsc.load_gather(idxp_v, [r[1], lanes])
        s2 = plsc.load_gather(idxp_v, [r[2], lanes])
        g0 = pltpu.async_copy(big_hbm.at[s0], rows_v.at[0:16], sem)
        g1 = pltpu.async_copy(big_hbm.at[s1], rows_v.at[16:32], sem)
        g2 = pltpu.async_copy(big_hbm.at[s2], rows_v.at[32:48], sem)
        cb1 = plsc.load_gather(idxp_v, [r[3], lanes])
        cb2 = plsc.load_gather(idxp_v, [r[4], lanes])
        cbn1 = plsc.load_gather(idxp_v, [r[5], lanes])
        cbn2 = plsc.load_gather(idxp_v, [r[6], lanes])
        g0.wait()
        g1.wait()
        g2.wait()

        jidx = jnp.minimum(lanes, 9)
        slot_w = 34 + (jidx >> 2)
        cb_w = (jidx & 3) << 5
        slot_x = jnp.full((16,), _SLOT_EXTRA, jnp.int32)
        slot_n1 = jnp.full((16,), 32, jnp.int32)
        slot_n2 = jnp.full((16,), 33, jnp.int32)
        lanes2 = lanes + 16

        acc1 = jnp.zeros((16,), jnp.float32)
        acc2 = jnp.zeros((16,), jnp.float32)
        acc3 = jnp.zeros((16,), jnp.float32)
        acc4 = jnp.zeros((16,), jnp.float32)
        for d in range(_EMBED):
            dvec = jnp.full((16,), d, jnp.int32)
            wv = plsc.load_gather(rows_v, [slot_w, cb_w + dvec])
            w3 = plsc.load_gather(rows_v, [slot_x, jnp.full((16,), _CB_W3 + d, jnp.int32)])
            v3 = plsc.load_gather(rows_v, [lanes, cb1 + dvec])
            v4 = plsc.load_gather(rows_v, [lanes2, cb2 + dvec])
            e1 = plsc.load_gather(rows_v, [slot_n1, cbn1 + dvec])
            e2 = plsc.load_gather(rows_v, [slot_n2, cbn2 + dvec])
            acc1 = acc1 + e1 * wv
            acc2 = acc2 + e2 * wv
            acc3 = acc3 + v3 * w3
            acc4 = acc4 + v4 * w3

        bv = plsc.load_gather(rows_v, [slot_x, _CB_BIAS + jidx])
        b3v = plsc.load_gather(rows_v, [slot_x, jnp.full((16,), _CB_BIAS + 10, jnp.int32)])
        bias = bv + b3v
        o_v[0:16] = acc1 + acc3 + bias
        o_v[16:32] = acc2 + acc4 + bias
        pltpu.sync_copy(o_v, o_hbm)


@functools.lru_cache(maxsize=1)
def _sc_call():
    return functools.partial(
        pl.kernel,
        mesh=plsc.VectorSubcoreMesh(core_axis_name="c", subcore_axis_name="s",
                                    num_cores=1, num_subcores=1),
        compiler_params=pltpu.CompilerParams(
            needs_layout_passes=False, use_tc_tiling_on_sc=False),
        out_type=[jax.ShapeDtypeStruct((32,), jnp.float32)],
        scratch_types=[
            pltpu.VMEM((8, _EMBED), jnp.int32),
            pltpu.VMEM((_NSLICE, 128), jnp.float32),
            pltpu.VMEM((32,), jnp.float32),
            pltpu.SemaphoreType.DMA,
        ],
    )(_sc_body)


def kernel(DPTD_name_1, DPTD_name_2, DPTD_word_1, DPTD_word_2,
           table, W, b, W3, b3):
    w1 = DPTD_word_1.astype(jnp.int32)
    w2 = DPTD_word_2.astype(jnp.int32)
    n1 = jnp.asarray(DPTD_name_1, jnp.int32)
    n2 = jnp.asarray(DPTD_name_2, jnp.int32)

    brow = jnp.concatenate([b, b3, jnp.zeros((9,), jnp.float32)]).reshape(1, _EMBED)
    big = jnp.pad(jnp.concatenate([table, W, W3, brow]), ((0, 0), (0, 12)))
    big = big.reshape(253, 128)

    pad6 = jnp.zeros((6,), jnp.int32)
    extras = jnp.concatenate([
        (n1 >> 2).reshape(1), (n2 >> 2).reshape(1),
        jnp.array([250, 251, 252], jnp.int32), jnp.zeros((11,), jnp.int32)])
    idxp = jnp.stack([
        jnp.concatenate([w1 >> 2, pad6]),
        jnp.concatenate([w2 >> 2, pad6]),
        extras,
        jnp.concatenate([(w1 & 3) << 5, pad6]),
        jnp.concatenate([(w2 & 3) << 5, pad6]),
        jnp.full((16,), (n1 & 3) << 5, jnp.int32),
        jnp.full((16,), (n2 & 3) << 5, jnp.int32),
        jnp.zeros((16,), jnp.int32),
    ])
    idxp = jnp.pad(idxp, ((0, 0), (0, _EMBED - 16)))

    (out,) = _sc_call()(big, idxp)
    return (out[0:10].reshape(1, 10), out[16:26].reshape(1, 10))


# speedup vs baseline: 1.0952x; 1.0952x over previous
import functools

import jax
import jax.numpy as jnp
from jax import lax
from jax.experimental import pallas as pl
from jax.experimental.pallas import tpu as pltpu
from jax.experimental.pallas import tpu_sc as plsc

_EMBED = 20
_NROWS = 22


def _sc_body(table_hbm, idxp_hbm, aux_hbm, o_hbm,
             idxp_v, rows_v, aux_v, o_v, sem, sem2):
    c = lax.axis_index("c")
    s = lax.axis_index("s")

    @pl.when(jnp.logical_and(c == 0, s == 0))
    def _():
        pltpu.sync_copy(idxp_hbm, idxp_v)
        aux_cp = pltpu.async_copy(aux_hbm, aux_v, sem2)
        lanes = lax.iota(jnp.int32, 16)
        r0 = jnp.zeros((16,), jnp.int32)
        w1vec = plsc.load_gather(idxp_v, [r0, lanes])
        w2vec = plsc.load_gather(idxp_v, [r0 + 1, lanes])
        nvec = plsc.load_gather(idxp_v, [r0 + 2, lanes])
        copies = []
        for k in range(10):
            copies.append(pltpu.async_copy(
                table_hbm.at[pl.ds(w1vec[k], 1), :],
                rows_v.at[pl.ds(k, 1), :], sem))
        for k in range(10):
            copies.append(pltpu.async_copy(
                table_hbm.at[pl.ds(w2vec[k], 1), :],
                rows_v.at[pl.ds(10 + k, 1), :], sem))
        copies.append(pltpu.async_copy(
            table_hbm.at[pl.ds(nvec[0], 1), :], rows_v.at[pl.ds(20, 1), :], sem))
        copies.append(pltpu.async_copy(
            table_hbm.at[pl.ds(nvec[1], 1), :], rows_v.at[pl.ds(21, 1), :], sem))
        aux_cp.wait()
        for cp in copies:
            cp.wait()
        jidx = jnp.minimum(lanes, 9)
        slot1 = jidx
        slot2 = 10 + jidx
        slot_n1 = jnp.full((16,), 20, jnp.int32)
        slot_n2 = jnp.full((16,), 21, jnp.int32)
        row_w3 = jnp.full((16,), 10, jnp.int32)
        row_b = jnp.full((16,), 11, jnp.int32)

        acc1 = jnp.zeros((16,), jnp.float32)
        acc2 = jnp.zeros((16,), jnp.float32)
        acc3 = jnp.zeros((16,), jnp.float32)
        acc4 = jnp.zeros((16,), jnp.float32)
        for d in range(_EMBED):
            dvec = jnp.full((16,), d, jnp.int32)
            wv = plsc.load_gather(aux_v, [jidx, dvec])
            w3 = plsc.load_gather(aux_v, [row_w3, dvec])
            v3 = plsc.load_gather(rows_v, [slot1, dvec])
            v4 = plsc.load_gather(rows_v, [slot2, dvec])
            e1 = plsc.load_gather(rows_v, [slot_n1, dvec])
            e2 = plsc.load_gather(rows_v, [slot_n2, dvec])
            acc1 = acc1 + e1 * wv
            acc2 = acc2 + e2 * wv
            acc3 = acc3 + v3 * w3
            acc4 = acc4 + v4 * w3

        bv = plsc.load_gather(aux_v, [row_b, jidx])
        b3v = plsc.load_gather(aux_v, [row_b, jnp.full((16,), 10, jnp.int32)])
        bias = bv + b3v
        o_v[0:16] = acc1 + acc3 + bias
        o_v[16:32] = acc2 + acc4 + bias
        pltpu.sync_copy(o_v, o_hbm)


@functools.lru_cache(maxsize=1)
def _sc_call():
    return functools.partial(
        pl.kernel,
        mesh=plsc.VectorSubcoreMesh(core_axis_name="c", subcore_axis_name="s",
                                    num_cores=1, num_subcores=1),
        compiler_params=pltpu.CompilerParams(
            needs_layout_passes=False, use_tc_tiling_on_sc=False),
        out_type=[jax.ShapeDtypeStruct((32,), jnp.float32)],
        scratch_types=[
            pltpu.VMEM((8, _EMBED), jnp.int32),
            pltpu.VMEM((_NROWS, _EMBED), jnp.float32),
            pltpu.VMEM((12, _EMBED), jnp.float32),
            pltpu.VMEM((32,), jnp.float32),
            pltpu.SemaphoreType.DMA,
            pltpu.SemaphoreType.DMA,
        ],
    )(_sc_body)


def kernel(DPTD_name_1, DPTD_name_2, DPTD_word_1, DPTD_word_2,
           table, W, b, W3, b3):
    w1 = DPTD_word_1.astype(jnp.int32)
    w2 = DPTD_word_2.astype(jnp.int32)
    pad6 = jnp.zeros((6,), jnp.int32)
    idxp = jnp.stack([
        jnp.concatenate([w1, pad6]),
        jnp.concatenate([w2, pad6]),
        jnp.concatenate([jnp.asarray(DPTD_name_1, jnp.int32).reshape(1),
                         jnp.asarray(DPTD_name_2, jnp.int32).reshape(1),
                         jnp.zeros((14,), jnp.int32)]),
        jnp.zeros((16,), jnp.int32),
        jnp.zeros((16,), jnp.int32),
        jnp.zeros((16,), jnp.int32),
        jnp.zeros((16,), jnp.int32),
        jnp.zeros((16,), jnp.int32),
    ])
    idxp = jnp.pad(idxp, ((0, 0), (0, _EMBED - 16)))
    aux = jnp.concatenate([
        W,
        W3,
        jnp.concatenate([b, b3, jnp.zeros((9,), jnp.float32)]).reshape(1, _EMBED),
    ])

    (out,) = _sc_call()(table, idxp, aux)
    return (out[0:10].reshape(1, 10), out[16:26].reshape(1, 10))


# speedup vs baseline: 1.1189x; 1.0216x over previous
import functools

import jax
import jax.numpy as jnp
from jax import lax
from jax.experimental import pallas as pl
from jax.experimental.pallas import tpu as pltpu
from jax.experimental.pallas import tpu_sc as plsc

_EMBED = 20
_NROWS = 22


def _sc_body(table_hbm, pack_hbm, o_hbm, pack_v, rows_v, o_v, sem):
    c = lax.axis_index("c")
    s = lax.axis_index("s")

    @pl.when(jnp.logical_and(c == 0, s == 0))
    def _():
        pltpu.sync_copy(pack_hbm, pack_v)
        lanes = lax.iota(jnp.int32, 16)
        r0 = jnp.zeros((16,), jnp.int32)
        i0a = plsc.bitcast(plsc.load_gather(pack_v, [r0, lanes]), jnp.int32)
        i0b = plsc.bitcast(plsc.load_gather(pack_v, [r0, lanes + 4]), jnp.int32)
        i1 = plsc.bitcast(plsc.load_gather(pack_v, [r0 + 1, lanes]), jnp.int32)
        copies = []
        for k in range(10):
            copies.append(pltpu.async_copy(
                table_hbm.at[pl.ds(i0a[k], 1), :],
                rows_v.at[pl.ds(k, 1), :], sem))
        for k in range(10):
            src = i0a[10 + k] if k < 6 else i0b[6 + k]
            copies.append(pltpu.async_copy(
                table_hbm.at[pl.ds(src, 1), :],
                rows_v.at[pl.ds(10 + k, 1), :], sem))
        copies.append(pltpu.async_copy(
            table_hbm.at[pl.ds(i1[0], 1), :], rows_v.at[pl.ds(20, 1), :], sem))
        copies.append(pltpu.async_copy(
            table_hbm.at[pl.ds(i1[1], 1), :], rows_v.at[pl.ds(21, 1), :], sem))

        jidx = jnp.minimum(lanes, 9)
        slot1 = jidx
        slot2 = 10 + jidx
        row_w3 = jnp.full((16,), 12, jnp.int32)
        row_b = jnp.full((16,), 13, jnp.int32)

        w3a = plsc.load_gather(pack_v, [row_w3, lanes])
        w3b = plsc.load_gather(pack_v, [row_w3, lanes + 4])
        bv = plsc.load_gather(pack_v, [row_b, jidx])
        b3a = plsc.load_gather(pack_v, [row_b, lanes])

        for cp in copies:
            cp.wait()

        e1a = plsc.load_gather(rows_v, [jnp.full((16,), 20, jnp.int32), lanes])
        e1b = plsc.load_gather(rows_v, [jnp.full((16,), 20, jnp.int32), lanes + 4])
        e2a = plsc.load_gather(rows_v, [jnp.full((16,), 21, jnp.int32), lanes])
        e2b = plsc.load_gather(rows_v, [jnp.full((16,), 21, jnp.int32), lanes + 4])

        acc1 = jnp.zeros((16,), jnp.float32)
        acc2 = jnp.zeros((16,), jnp.float32)
        acc3 = jnp.zeros((16,), jnp.float32)
        acc4 = jnp.zeros((16,), jnp.float32)
        for d in range(_EMBED):
            dvec = jnp.full((16,), d, jnp.int32)
            wv = plsc.load_gather(pack_v, [2 + jidx, dvec])
            v3 = plsc.load_gather(rows_v, [slot1, dvec])
            v4 = plsc.load_gather(rows_v, [slot2, dvec])
            w3 = w3a[d] if d < 16 else w3b[d - 4]
            e1 = e1a[d] if d < 16 else e1b[d - 4]
            e2 = e2a[d] if d < 16 else e2b[d - 4]
            acc1 = acc1 + e1 * wv
            acc2 = acc2 + e2 * wv
            acc3 = acc3 + v3 * w3
            acc4 = acc4 + v4 * w3

        bias = bv + b3a[10]
        o_v[0:16] = acc1 + acc3 + bias
        o_v[16:32] = acc2 + acc4 + bias
        pltpu.sync_copy(o_v, o_hbm)


@functools.lru_cache(maxsize=1)
def _sc_call():
    return functools.partial(
        pl.kernel,
        mesh=plsc.VectorSubcoreMesh(core_axis_name="c", subcore_axis_name="s",
                                    num_cores=1, num_subcores=1),
        compiler_params=pltpu.CompilerParams(
            needs_layout_passes=False, use_tc_tiling_on_sc=False),
        out_type=[jax.ShapeDtypeStruct((32,), jnp.float32)],
        scratch_types=[
            pltpu.VMEM((14, _EMBED), jnp.float32),
            pltpu.VMEM((_NROWS, _EMBED), jnp.float32),
            pltpu.VMEM((32,), jnp.float32),
            pltpu.SemaphoreType.DMA,
        ],
    )(_sc_body)


def kernel(DPTD_name_1, DPTD_name_2, DPTD_word_1, DPTD_word_2,
           table, W, b, W3, b3):
    wf1 = lax.bitcast_convert_type(DPTD_word_1.astype(jnp.int32), jnp.float32)
    wf2 = lax.bitcast_convert_type(DPTD_word_2.astype(jnp.int32), jnp.float32)
    nf = lax.bitcast_convert_type(
        jnp.stack([jnp.asarray(DPTD_name_1, jnp.int32),
                   jnp.asarray(DPTD_name_2, jnp.int32)]), jnp.float32)
    pack = jnp.concatenate([
        wf1, wf2,
        nf, jnp.zeros((18,), jnp.float32),
        W.reshape(-1),
        W3.reshape(-1),
        b, b3, jnp.zeros((9,), jnp.float32),
    ]).reshape(14, _EMBED)

    (out,) = _sc_call()(table, pack)
    return (out[0:10].reshape(1, 10), out[16:26].reshape(1, 10))


# speedup vs baseline: 1.1252x; 1.0057x over previous
import functools

import jax
import jax.numpy as jnp
from jax import lax
from jax.experimental import pallas as pl
from jax.experimental.pallas import tpu as pltpu
from jax.experimental.pallas import tpu_sc as plsc

_EMBED = 20
_NROWS = 22


def _sc_body(table_hbm, pack_hbm, o_hbm, pack_v, rows_v, o_v, sem):
    c = lax.axis_index("c")
    s = lax.axis_index("s")

    @pl.when(jnp.logical_and(c == 0, s == 0))
    def _():
        pltpu.sync_copy(pack_hbm, pack_v)
        lanes = lax.iota(jnp.int32, 16)
        r0 = jnp.zeros((16,), jnp.int32)
        i0a = plsc.bitcast(plsc.load_gather(pack_v, [r0, lanes]), jnp.int32)
        i0b = plsc.bitcast(plsc.load_gather(pack_v, [r0, lanes + 4]), jnp.int32)
        i1 = plsc.bitcast(plsc.load_gather(pack_v, [r0 + 1, lanes]), jnp.int32)
        copies = []
        for k in range(10):
            copies.append(pltpu.async_copy(
                table_hbm.at[pl.ds(i0a[k], 1), :],
                rows_v.at[pl.ds(k, 1), :], sem))
        for k in range(10):
            src = i0a[10 + k] if k < 6 else i0b[6 + k]
            copies.append(pltpu.async_copy(
                table_hbm.at[pl.ds(src, 1), :],
                rows_v.at[pl.ds(10 + k, 1), :], sem))
        copies.append(pltpu.async_copy(
            table_hbm.at[pl.ds(i1[0], 1), :], rows_v.at[pl.ds(20, 1), :], sem))
        copies.append(pltpu.async_copy(
            table_hbm.at[pl.ds(i1[1], 1), :], rows_v.at[pl.ds(21, 1), :], sem))

        jidx = jnp.minimum(lanes, 9)
        slot1 = jidx
        slot2 = 10 + jidx
        row_w3 = jnp.full((16,), 12, jnp.int32)
        row_b = jnp.full((16,), 13, jnp.int32)

        w3a = plsc.load_gather(pack_v, [row_w3, lanes])
        w3b = plsc.load_gather(pack_v, [row_w3, lanes + 4])
        bv = plsc.load_gather(pack_v, [row_b, jidx])
        b3a = plsc.load_gather(pack_v, [row_b, lanes])

        for cp in copies:
            cp.wait()

        e1a = plsc.load_gather(rows_v, [jnp.full((16,), 20, jnp.int32), lanes])
        e1b = plsc.load_gather(rows_v, [jnp.full((16,), 20, jnp.int32), lanes + 4])
        e2a = plsc.load_gather(rows_v, [jnp.full((16,), 21, jnp.int32), lanes])
        e2b = plsc.load_gather(rows_v, [jnp.full((16,), 21, jnp.int32), lanes + 4])

        acc1 = jnp.zeros((16,), jnp.float32)
        acc2 = jnp.zeros((16,), jnp.float32)
        acc3 = jnp.zeros((16,), jnp.float32)
        acc4 = jnp.zeros((16,), jnp.float32)
        for d in range(_EMBED):
            dvec = jnp.full((16,), d, jnp.int32)
            wv = plsc.load_gather(pack_v, [2 + jidx, dvec])
            v3 = plsc.load_gather(rows_v, [slot1, dvec])
            v4 = plsc.load_gather(rows_v, [slot2, dvec])
            w3 = w3a[d] if d < 16 else w3b[d - 4]
            e1 = e1a[d] if d < 16 else e1b[d - 4]
            e2 = e2a[d] if d < 16 else e2b[d - 4]
            acc1 = acc1 + e1 * wv
            acc2 = acc2 + e2 * wv
            acc3 = acc3 + v3 * w3
            acc4 = acc4 + v4 * w3

        bias = bv + b3a[10]
        o_v[0:16] = acc1 + acc3 + bias
        o_v[16:32] = acc2 + acc4 + bias
        pltpu.sync_copy(o_v, o_hbm)


@functools.lru_cache(maxsize=1)
def _sc_call():
    return functools.partial(
        pl.kernel,
        mesh=plsc.VectorSubcoreMesh(core_axis_name="c", subcore_axis_name="s",
                                    num_cores=1, num_subcores=1),
        compiler_params=pltpu.CompilerParams(
            needs_layout_passes=False, use_tc_tiling_on_sc=True),
        out_type=[jax.ShapeDtypeStruct((32,), jnp.float32)],
        scratch_types=[
            pltpu.VMEM((14, _EMBED), jnp.float32),
            pltpu.VMEM((_NROWS, _EMBED), jnp.float32),
            pltpu.VMEM((32,), jnp.float32),
            pltpu.SemaphoreType.DMA,
        ],
    )(_sc_body)


def kernel(DPTD_name_1, DPTD_name_2, DPTD_word_1, DPTD_word_2,
           table, W, b, W3, b3):
    wf1 = lax.bitcast_convert_type(DPTD_word_1.astype(jnp.int32), jnp.float32)
    wf2 = lax.bitcast_convert_type(DPTD_word_2.astype(jnp.int32), jnp.float32)
    nf = lax.bitcast_convert_type(
        jnp.stack([jnp.asarray(DPTD_name_1, jnp.int32),
                   jnp.asarray(DPTD_name_2, jnp.int32)]), jnp.float32)
    pack = jnp.concatenate([
        wf1, wf2,
        nf, jnp.zeros((18,), jnp.float32),
        W.reshape(-1),
        W3.reshape(-1),
        b, b3, jnp.zeros((9,), jnp.float32),
    ]).reshape(14, _EMBED)

    (out,) = _sc_call()(table, pack)
    return (out[0:10].reshape(1, 10), out[16:26].reshape(1, 10))


# speedup vs baseline: 1.1802x; 1.0489x over previous
import functools

import jax
import jax.numpy as jnp
from jax import lax
from jax.experimental import pallas as pl
from jax.experimental.pallas import tpu as pltpu
from jax.experimental.pallas import tpu_sc as plsc

_EMBED = 20
_NROWS = 22


def _sc_body(table_hbm, pack_hbm, o1_hbm, o2_hbm, pack_v, rows_v, o_v, sem):
    c = lax.axis_index("c")
    s = lax.axis_index("s")

    @pl.when(jnp.logical_and(c == 0, s == 0))
    def _():
        pltpu.sync_copy(pack_hbm, pack_v)
        lanes = lax.iota(jnp.int32, 16)
        r0 = jnp.zeros((16,), jnp.int32)
        i0a = plsc.bitcast(plsc.load_gather(pack_v, [r0, lanes]), jnp.int32)
        i0b = plsc.bitcast(plsc.load_gather(pack_v, [r0, lanes + 4]), jnp.int32)
        i1 = plsc.bitcast(plsc.load_gather(pack_v, [r0 + 1, lanes]), jnp.int32)
        copies = []
        for k in range(10):
            copies.append(pltpu.async_copy(
                table_hbm.at[pl.ds(i0a[k], 1), :],
                rows_v.at[pl.ds(k, 1), :], sem))
        for k in range(10):
            src = i0a[10 + k] if k < 6 else i0b[6 + k]
            copies.append(pltpu.async_copy(
                table_hbm.at[pl.ds(src, 1), :],
                rows_v.at[pl.ds(10 + k, 1), :], sem))
        copies.append(pltpu.async_copy(
            table_hbm.at[pl.ds(i1[0], 1), :], rows_v.at[pl.ds(20, 1), :], sem))
        copies.append(pltpu.async_copy(
            table_hbm.at[pl.ds(i1[1], 1), :], rows_v.at[pl.ds(21, 1), :], sem))

        jidx = jnp.minimum(lanes, 9)
        slot1 = jidx
        slot2 = 10 + jidx
        row_w3 = jnp.full((16,), 12, jnp.int32)
        row_b = jnp.full((16,), 13, jnp.int32)

        w3a = plsc.load_gather(pack_v, [row_w3, lanes])
        w3b = plsc.load_gather(pack_v, [row_w3, lanes + 4])
        bv = plsc.load_gather(pack_v, [row_b, jidx])
        b3a = plsc.load_gather(pack_v, [row_b, lanes])

        for cp in copies:
            cp.wait()

        e1a = plsc.load_gather(rows_v, [jnp.full((16,), 20, jnp.int32), lanes])
        e1b = plsc.load_gather(rows_v, [jnp.full((16,), 20, jnp.int32), lanes + 4])
        e2a = plsc.load_gather(rows_v, [jnp.full((16,), 21, jnp.int32), lanes])
        e2b = plsc.load_gather(rows_v, [jnp.full((16,), 21, jnp.int32), lanes + 4])

        acc1 = jnp.zeros((16,), jnp.float32)
        acc2 = jnp.zeros((16,), jnp.float32)
        acc3 = jnp.zeros((16,), jnp.float32)
        acc4 = jnp.zeros((16,), jnp.float32)
        for d in range(_EMBED):
            dvec = jnp.full((16,), d, jnp.int32)
            wv = plsc.load_gather(pack_v, [2 + jidx, dvec])
            v3 = plsc.load_gather(rows_v, [slot1, dvec])
            v4 = plsc.load_gather(rows_v, [slot2, dvec])
            w3 = w3a[d] if d < 16 else w3b[d - 4]
            e1 = e1a[d] if d < 16 else e1b[d - 4]
            e2 = e2a[d] if d < 16 else e2b[d - 4]
            acc1 = acc1 + e1 * wv
            acc2 = acc2 + e2 * wv
            acc3 = acc3 + v3 * w3
            acc4 = acc4 + v4 * w3

        bias = bv + b3a[10]
        o_v[0:16] = acc1 + acc3 + bias
        o_v[16:32] = acc2 + acc4 + bias
        pltpu.sync_copy(o_v.at[0:10], o1_hbm.at[0])
        pltpu.sync_copy(o_v.at[16:26], o2_hbm.at[0])


@functools.lru_cache(maxsize=1)
def _sc_call():
    return functools.partial(
        pl.kernel,
        mesh=plsc.VectorSubcoreMesh(core_axis_name="c", subcore_axis_name="s",
                                    num_cores=1, num_subcores=1),
        compiler_params=pltpu.CompilerParams(
            needs_layout_passes=False, use_tc_tiling_on_sc=True),
        out_type=[jax.ShapeDtypeStruct((1, 10), jnp.float32),
                  jax.ShapeDtypeStruct((1, 10), jnp.float32)],
        scratch_types=[
            pltpu.VMEM((14, _EMBED), jnp.float32),
            pltpu.VMEM((_NROWS, _EMBED), jnp.float32),
            pltpu.VMEM((32,), jnp.float32),
            pltpu.SemaphoreType.DMA,
        ],
    )(_sc_body)


def kernel(DPTD_name_1, DPTD_name_2, DPTD_word_1, DPTD_word_2,
           table, W, b, W3, b3):
    wf1 = lax.bitcast_convert_type(DPTD_word_1.astype(jnp.int32), jnp.float32)
    wf2 = lax.bitcast_convert_type(DPTD_word_2.astype(jnp.int32), jnp.float32)
    nf = lax.bitcast_convert_type(
        jnp.stack([jnp.asarray(DPTD_name_1, jnp.int32),
                   jnp.asarray(DPTD_name_2, jnp.int32)]), jnp.float32)
    pack = jnp.concatenate([
        wf1, wf2,
        nf, jnp.zeros((18,), jnp.float32),
        W.reshape(-1),
        W3.reshape(-1),
        b, b3, jnp.zeros((9,), jnp.float32),
    ]).reshape(14, _EMBED)

    r1, r2 = _sc_call()(table, pack)
    return (r1, r2)
